# baseline (speedup 1.0000x reference)
import functools
import math

import jax
import jax.numpy as jnp
from jax import lax
from jax.experimental import pallas as pl
from jax.experimental.pallas import tpu as pltpu
from jax.experimental.pallas import tpu_sc as plsc

D_MODEL = 64
SCALE = math.sqrt(D_MODEL)

_INFO = plsc.get_sparse_core_info()
NC = _INFO.num_cores
NS = _INFO.num_subcores
NW = NC * NS
IDX_W = 128
G = 5
C = G * IDX_W
R_UNROLL = 8


def _build(n_rows: int):
    rows_per_w = n_rows // NW
    n_chunks = rows_per_w // G
    n_pair = n_chunks // 2
    B = n_rows * IDX_W

    mesh = plsc.VectorSubcoreMesh(core_axis_name="c", subcore_axis_name="s")

    @functools.partial(
        pl.kernel,
        mesh=mesh,
        out_type=jax.ShapeDtypeStruct((B, D_MODEL), jnp.float32),
        scratch_types=[
            pltpu.VMEM((rows_per_w, IDX_W), jnp.int32),
            pltpu.VMEM((C, D_MODEL), jnp.float32),
            pltpu.VMEM((C, D_MODEL), jnp.float32),
            pltpu.SemaphoreType.DMA,
            pltpu.SemaphoreType.DMA,
            pltpu.SemaphoreType.DMA,
            pltpu.SemaphoreType.DMA,
        ],
        compiler_params=pltpu.CompilerParams(use_tc_tiling_on_sc=False),
    )
    def emb(idx_hbm, table_hbm, out_hbm, idx_v, rows0, rows1,
            gsem0, gsem1, wsem0, wsem1):
        wid = lax.axis_index("s") * NC + lax.axis_index("c")
        w_row0 = wid * rows_per_w
        rows = (rows0, rows1)
        gsem = (gsem0, gsem1)
        wsem = (wsem0, wsem1)

        pltpu.sync_copy(idx_hbm.at[pl.ds(w_row0, rows_per_w)], idx_v)

        def fire(c, b):
            for j in range(G):
                pltpu.async_copy(
                    table_hbm.at[idx_v.at[c * G + j]],
                    rows[b].at[pl.ds(j * IDX_W, IDX_W)],
                    gsem[b],
                )

        def gather_wait(b):
            pltpu.make_async_copy(
                out_hbm.at[pl.ds(0, C)], rows[b], gsem[b]).wait()

        def wb_start(c, b):
            pltpu.async_copy(
                rows[b], out_hbm.at[pl.ds((w_row0 + c * G) * IDX_W, C)],
                wsem[b])

        def wb_wait(b):
            pltpu.make_async_copy(
                rows[b], out_hbm.at[pl.ds(0, C)], wsem[b]).wait()

        def scale(b):
            def body(i, carry):
                base = i * R_UNROLL
                for r in range(R_UNROLL):
                    for l in range(D_MODEL // 16):
                        sl = pl.ds(l * 16, 16)
                        rows[b][base + r, sl] = rows[b][base + r, sl] * SCALE
                return carry
            lax.fori_loop(0, C // R_UNROLL, body, 0)

        fire(0, 0)

        def pair(p, carry):
            c0 = 2 * p
            gather_wait(0)
            scale(0)

            @pl.when(p > 0)
            def _():
                wb_wait(1)
            fire(c0 + 1, 1)
            wb_start(c0, 0)

            gather_wait(1)
            scale(1)

            wb_wait(0)

            @pl.when(p < n_pair - 1)
            def _():
                fire(c0 + 2, 0)
            wb_start(c0 + 1, 1)
            return carry

        lax.fori_loop(0, n_pair, pair, 0)
        wb_wait(1)

    return emb


@jax.jit
def kernel(x, table):
    B0, S = x.shape
    B = B0 * S
    idx2 = x.reshape(B // IDX_W, IDX_W).astype(jnp.int32)
    out = _build(B // IDX_W)(idx2, table)
    return out.reshape(B0, S, D_MODEL)
